# baseline (speedup 1.0000x reference)
import functools

import jax
import jax.numpy as jnp
from jax import lax
from jax.experimental import pallas as pl
from jax.experimental.pallas import tpu as pltpu
from jax.experimental.pallas import tpu_sc as plsc

N = 10000
E = 320000
D_IN = 128
EMB = 256
N_ACT = 64
B = 1024

NC = 2
NS = 16
NW = NC * NS

QS = 2560
NQ = 4
C = 128
NCHUNK_Q = 48
SLAB = 8
NSLAB = NCHUNK_Q // SLAB
CAP = NS * NCHUNK_Q * C
RB = 2

ACC_N = 3072
ZROWS = ACC_N // NS
WROWS = QS // NS
N_OUT_PAD = NQ * QS

_P = jax.lax.Precision.HIGHEST


def _dot(a, b):
    return jax.lax.dot(a, b, precision=_P, preferred_element_type=jnp.float32)


R = 1000
GRID_N = N // R


def _embed_body(x_ref, w_ref, b_ref, y_ref, st_ref):
    i = pl.program_id(0)
    y = _dot(x_ref[...], w_ref[...]) + b_ref[...]
    y_ref[...] = y
    s1 = jnp.sum(y, axis=0, keepdims=True)
    s2 = jnp.sum(y * y, axis=0, keepdims=True)
    acc = jnp.concatenate([s1, s2], axis=0)

    @pl.when(i == 0)
    def _():
        st_ref[...] = jnp.zeros_like(st_ref)

    st_ref[...] += acc


def _embed_call(x, w, b):
    return pl.pallas_call(
        _embed_body,
        grid=(GRID_N,),
        in_specs=[
            pl.BlockSpec((R, D_IN), lambda i: (i, 0)),
            pl.BlockSpec((D_IN, EMB), lambda i: (0, 0)),
            pl.BlockSpec((1, EMB), lambda i: (0, 0)),
        ],
        out_specs=[
            pl.BlockSpec((R, EMB), lambda i: (i, 0)),
            pl.BlockSpec((2, EMB), lambda i: (0, 0)),
        ],
        out_shape=[
            jax.ShapeDtypeStruct((N, EMB), jnp.float32),
            jax.ShapeDtypeStruct((2, EMB), jnp.float32),
        ],
    )(x, w, b)


def _bn_body(y_ref, st_ref, g_ref, bt_ref, xf_ref):
    st = st_ref[...]
    mu = st[0:1, :] * (1.0 / N)
    var = st[1:2, :] * (1.0 / N) - mu * mu
    xf_ref[...] = ((y_ref[...] - mu) * jax.lax.rsqrt(var + 1e-5) * g_ref[...]
                   + bt_ref[...])


def _bn_call(y, st, gamma, beta):
    return pl.pallas_call(
        _bn_body,
        grid=(GRID_N,),
        in_specs=[
            pl.BlockSpec((R, EMB), lambda i: (i, 0)),
            pl.BlockSpec((2, EMB), lambda i: (0, 0)),
            pl.BlockSpec((1, EMB), lambda i: (0, 0)),
            pl.BlockSpec((1, EMB), lambda i: (0, 0)),
        ],
        out_specs=pl.BlockSpec((R, EMB), lambda i: (i, 0)),
        out_shape=jax.ShapeDtypeStruct((N, EMB), jnp.float32),
    )(y, st, gamma, beta)


def _gin0_body(xf_ref, a_ref, w1_ref, b1_ref, w2_ref, b2_ref, eps_ref, hf_ref):
    z = (1.0 + eps_ref[0, 0]) * xf_ref[...] + a_ref[...]
    t = jnp.maximum(_dot(z, w1_ref[...]) + b1_ref[...], 0.0)
    hf_ref[...] = _dot(t, w2_ref[...]) + b2_ref[...]


def _gin0_call(xf, agg, w1, b1, w2, b2, eps):
    return pl.pallas_call(
        _gin0_body,
        grid=(GRID_N,),
        in_specs=[
            pl.BlockSpec((R, EMB), lambda i: (i, 0)),
            pl.BlockSpec((R, EMB), lambda i: (i, 0)),
            pl.BlockSpec((EMB, EMB), lambda i: (0, 0)),
            pl.BlockSpec((1, EMB), lambda i: (0, 0)),
            pl.BlockSpec((EMB, EMB), lambda i: (0, 0)),
            pl.BlockSpec((1, EMB), lambda i: (0, 0)),
            pl.BlockSpec((1, 1), lambda i: (0, 0)),
        ],
        out_specs=pl.BlockSpec((R, EMB), lambda i: (i, 0)),
        out_shape=jax.ShapeDtypeStruct((N, EMB), jnp.float32),
    )(xf, agg, w1, b1, w2, b2, eps)


def _gin_body(xf_ref, hf_ref, ax_ref, ah_ref,
              w1_ref, b1_ref, w2_ref, b2_ref, eps_ref, hfo_ref):
    e = 1.0 + eps_ref[0, 0]
    za = e * xf_ref[...] + ax_ref[...]
    zb = e * hf_ref[...] + ah_ref[...]
    t = jnp.maximum(
        _dot(za, w1_ref[0:EMB, :]) + _dot(zb, w1_ref[EMB:, :]) + b1_ref[...], 0.0)
    hfo_ref[...] = _dot(t, w2_ref[...]) + b2_ref[...]


def _gin_call(xf, hf, ax, ah, w1, b1, w2, b2, eps):
    return pl.pallas_call(
        _gin_body,
        grid=(GRID_N,),
        in_specs=[
            pl.BlockSpec((R, EMB), lambda i: (i, 0)),
            pl.BlockSpec((R, EMB), lambda i: (i, 0)),
            pl.BlockSpec((R, EMB), lambda i: (i, 0)),
            pl.BlockSpec((R, EMB), lambda i: (i, 0)),
            pl.BlockSpec((2 * EMB, EMB), lambda i: (0, 0)),
            pl.BlockSpec((1, EMB), lambda i: (0, 0)),
            pl.BlockSpec((EMB, EMB), lambda i: (0, 0)),
            pl.BlockSpec((1, EMB), lambda i: (0, 0)),
            pl.BlockSpec((1, 1), lambda i: (0, 0)),
        ],
        out_specs=pl.BlockSpec((R, EMB), lambda i: (i, 0)),
        out_shape=jax.ShapeDtypeStruct((N, EMB), jnp.float32),
    )(xf, hf, ax, ah, w1, b1, w2, b2, eps)


def _head_body(xo_ref, w0_ref, b0_ref, w1_ref, b1_ref, w2_ref, b2_ref,
               m_ref, s_ref, l_ref):
    xo = xo_ref[...]
    o1 = _dot(xo, w0_ref[...]) + b0_ref[...]
    o2 = _dot(o1, w1_ref[0:EMB, :]) + _dot(xo, w1_ref[EMB:, :]) + b1_ref[...]
    o3 = _dot(o2, w2_ref[0:EMB, :]) + _dot(xo, w2_ref[EMB:, :]) + b2_ref[...]
    logits = jnp.where(m_ref[...] > 0.5, o3, -1.0e6)
    m = jnp.max(logits, axis=1, keepdims=True)
    ssum = jnp.sum(jnp.exp(logits - m), axis=1, keepdims=True)
    idx = lax.broadcasted_iota(jnp.int32, logits.shape, 1)
    samp = jnp.min(jnp.where(logits == m, idx, N_ACT), axis=1, keepdims=True)
    s_ref[...] = samp
    l_ref[...] = -jnp.log(ssum)


def _head_call(xo, w0, b0, w1, b1, w2, b2, maskf):
    return pl.pallas_call(
        _head_body,
        grid=(1,),
        in_specs=[
            pl.BlockSpec((B, EMB), lambda i: (0, 0)),
            pl.BlockSpec((EMB, EMB), lambda i: (0, 0)),
            pl.BlockSpec((1, EMB), lambda i: (0, 0)),
            pl.BlockSpec((2 * EMB, EMB), lambda i: (0, 0)),
            pl.BlockSpec((1, EMB), lambda i: (0, 0)),
            pl.BlockSpec((2 * EMB, N_ACT), lambda i: (0, 0)),
            pl.BlockSpec((1, N_ACT), lambda i: (0, 0)),
            pl.BlockSpec((B, N_ACT), lambda i: (0, 0)),
        ],
        out_specs=[
            pl.BlockSpec((B, 1), lambda i: (0, 0)),
            pl.BlockSpec((B, 1), lambda i: (0, 0)),
        ],
        out_shape=[
            jax.ShapeDtypeStruct((B, 1), jnp.int32),
            jax.ShapeDtypeStruct((B, 1), jnp.float32),
        ],
    )(xo, w0, b0, w1, b1, w2, b2, maskf)


@functools.cache
def _get_segsum():
    mesh = plsc.VectorSubcoreMesh(core_axis_name="c", subcore_axis_name="s",
                                  num_cores=NC, num_subcores=NS)
    return functools.partial(
        pl.kernel,
        mesh=mesh,
        out_type=jax.ShapeDtypeStruct((2 * N_OUT_PAD, 128), jnp.float32),
        scratch_types=[
            pltpu.VMEM((SLAB, C), jnp.int32),
            pltpu.VMEM((2 * SLAB, C), jnp.int32),
            pltpu.VMEM((RB, 2 * C, 128), jnp.float32),
            pltpu.VMEM_SHARED((2 * ACC_N, 128), jnp.float32),
            [pltpu.SemaphoreType.DMA] * RB,
        ],
    )(_segsum_body)


def _segsum_body(h_hbm, src_hbm, dstl_hbm, zeros_hbm, out_hbm,
                 src_v, dst_v, rows_v, acc_sh, gsems):
    c = lax.axis_index("c")
    s = lax.axis_index("s")
    zrows = 2 * ACC_N // NS

    for q in range(2):
        grp = c * 2 + q

        pltpu.sync_copy(zeros_hbm, acc_sh.at[pl.ds(s * zrows, zrows)])
        plsc.subcore_barrier()

        @pl.loop(0, NSLAB)
        def _slab(t):
            pltpu.sync_copy(src_hbm.at[grp, s, pl.ds(t * SLAB, SLAB)], src_v)
            pltpu.sync_copy(dstl_hbm.at[grp, s, pl.ds(t * 2 * SLAB, 2 * SLAB)],
                            dst_v)
            pltpu.async_copy(h_hbm.at[src_v.at[0]],
                             rows_v.at[0].reshape(C, EMB), gsems[0])
            pltpu.async_copy(h_hbm.at[src_v.at[1]],
                             rows_v.at[1].reshape(C, EMB), gsems[1])

            @pl.loop(0, SLAB, step=RB)
            def _grp_loop(g):
                for b in range(RB):
                    j = g + b
                    pltpu.make_async_copy(h_hbm.at[src_v.at[j]],
                                          rows_v.at[b].reshape(C, EMB),
                                          gsems[b]).wait()
                    for k in range(2):
                        pltpu.sync_copy(rows_v.at[b, pl.ds(k * C, C)],
                                        acc_sh.at[dst_v.at[2 * j + k]],
                                        add=True)

                    @pl.when(j + RB < SLAB)
                    def _():
                        pltpu.async_copy(h_hbm.at[src_v.at[j + RB]],
                                         rows_v.at[b].reshape(C, EMB),
                                         gsems[b])

        plsc.subcore_barrier()

        pltpu.sync_copy(acc_sh.at[pl.ds(s * 2 * WROWS, 2 * WROWS)],
                        out_hbm.at[pl.ds(2 * (grp * QS) + s * 2 * WROWS,
                                         2 * WROWS)])
        plsc.subcore_barrier()


B_PER_W = B // NW


@functools.cache
def _get_center_gather():
    mesh = plsc.VectorSubcoreMesh(core_axis_name="c", subcore_axis_name="s",
                                  num_cores=NC, num_subcores=NS)
    return functools.partial(
        pl.kernel,
        mesh=mesh,
        out_type=jax.ShapeDtypeStruct((B, EMB), jnp.float32),
        scratch_types=[
            pltpu.VMEM((B_PER_W,), jnp.int32),
            pltpu.VMEM((B_PER_W, EMB), jnp.float32),
            pltpu.SemaphoreType.DMA,
        ],
    )(_center_gather_body)


def _center_gather_body(h_hbm, idx_hbm, out_hbm, idx_v, rows_v, sem):
    wid = lax.axis_index("s") * NC + lax.axis_index("c")
    base = wid * B_PER_W
    pltpu.sync_copy(idx_hbm.at[pl.ds(base, B_PER_W)], idx_v)
    pltpu.async_copy(h_hbm.at[idx_v], rows_v, sem).wait()
    pltpu.sync_copy(rows_v, out_hbm.at[pl.ds(base, B_PER_W)])


def _segsum_kernel(h, src_r, dstl_r, zeros_init):
    out = _get_segsum()(h, src_r, dstl_r, zeros_init)
    return out.reshape(N_OUT_PAD, EMB)


def _center_gather_kernel(h, idx):
    return _get_center_gather()(h, idx)


def kernel(x, edge_index, center_node_index, mask, params):
    src = edge_index[0]
    dst = edge_index[1]

    grp = dst // QS
    cum = jnp.cumsum(jax.nn.one_hot(grp, NQ, dtype=jnp.int32), axis=0)
    rank = jnp.take_along_axis(cum, grp[:, None], axis=1)[:, 0] - 1
    pos = grp * CAP + rank
    garbage = QS + (jnp.arange(NQ * CAP, dtype=jnp.int32) % (ACC_N - QS))
    src_r = (jnp.zeros((NQ * CAP,), jnp.int32).at[pos].set(src)
             .reshape(NQ, NS, NCHUNK_Q, C))
    dstl = garbage.at[pos].set(dst - grp * QS)
    dstl_r = (jnp.stack([2 * dstl, 2 * dstl + 1], axis=-1)
              .reshape(NQ, NS, 2 * NCHUNK_Q, C))
    zeros_init = jnp.zeros((2 * ACC_N // NS, 128), jnp.float32)

    p = params
    be = p["embed"]["b"].reshape(1, EMB)
    y, st = _embed_call(x, p["embed"]["W"], be)
    xf = _bn_call(y, st, p["bn"]["gamma"].reshape(1, EMB),
                  p["bn"]["beta"].reshape(1, EMB))

    agg_x = _segsum_kernel(xf, src_r, dstl_r, zeros_init)

    g0 = p["gins"][0]
    h0f = _gin0_call(xf, agg_x[:N],
                     g0["lin1"]["W"], g0["lin1"]["b"].reshape(1, EMB),
                     g0["lin2"]["W"], g0["lin2"]["b"].reshape(1, EMB),
                     g0["eps"].reshape(1, 1))

    agg_h0 = _segsum_kernel(h0f, src_r, dstl_r, zeros_init)

    g1 = p["gins"][1]
    h1f = _gin_call(xf, h0f, agg_x[:N], agg_h0[:N],
                    g1["lin1"]["W"], g1["lin1"]["b"].reshape(1, EMB),
                    g1["lin2"]["W"], g1["lin2"]["b"].reshape(1, EMB),
                    g1["eps"].reshape(1, 1))

    agg_h1 = _segsum_kernel(h1f, src_r, dstl_r, zeros_init)

    g2 = p["gins"][2]
    h2f = _gin_call(xf, h1f, agg_x[:N], agg_h1[:N],
                    g2["lin1"]["W"], g2["lin1"]["b"].reshape(1, EMB),
                    g2["lin2"]["W"], g2["lin2"]["b"].reshape(1, EMB),
                    g2["eps"].reshape(1, 1))

    xo = _center_gather_kernel(h2f, center_node_index)

    o0, o1, o2 = p["outs"]
    s2d, l2d = _head_call(xo, o0["W"], o0["b"].reshape(1, EMB),
                          o1["W"], o1["b"].reshape(1, EMB),
                          o2["W"], o2["b"].reshape(1, N_ACT),
                          mask.astype(jnp.float32))
    return s2d[:, 0], l2d[:, 0]


# speedup vs baseline: 1.1878x; 1.1878x over previous
import functools

import jax
import jax.numpy as jnp
from jax import lax
from jax.experimental import pallas as pl
from jax.experimental.pallas import tpu as pltpu
from jax.experimental.pallas import tpu_sc as plsc

N = 10000
E = 320000
D_IN = 128
EMB = 256
N_ACT = 64
B = 1024

NC = 2
NS = 16
NW = NC * NS

QS = 2560
NQ = 4
C = 128
NCHUNK_Q = 48
SLAB = 8
NSLAB = NCHUNK_Q // SLAB
CAP = NS * NCHUNK_Q * C
RB = 2

ACC_N = 3072
ZROWS = ACC_N // NS
WROWS = QS // NS
N_OUT_PAD = NQ * QS

def _dot(a, b):
    return jax.lax.dot(a, b, preferred_element_type=jnp.float32)


R = 1000
GRID_N = N // R


def _embed_body(x_ref, w_ref, b_ref, y_ref, st_ref):
    i = pl.program_id(0)
    y = _dot(x_ref[...], w_ref[...]) + b_ref[...]
    y_ref[...] = y
    s1 = jnp.sum(y, axis=0, keepdims=True)
    s2 = jnp.sum(y * y, axis=0, keepdims=True)
    acc = jnp.concatenate([s1, s2], axis=0)

    @pl.when(i == 0)
    def _():
        st_ref[...] = jnp.zeros_like(st_ref)

    st_ref[...] += acc


def _embed_call(x, w, b):
    return pl.pallas_call(
        _embed_body,
        grid=(GRID_N,),
        in_specs=[
            pl.BlockSpec((R, D_IN), lambda i: (i, 0)),
            pl.BlockSpec((D_IN, EMB), lambda i: (0, 0)),
            pl.BlockSpec((1, EMB), lambda i: (0, 0)),
        ],
        out_specs=[
            pl.BlockSpec((R, EMB), lambda i: (i, 0)),
            pl.BlockSpec((2, EMB), lambda i: (0, 0)),
        ],
        out_shape=[
            jax.ShapeDtypeStruct((N, EMB), jnp.float32),
            jax.ShapeDtypeStruct((2, EMB), jnp.float32),
        ],
    )(x, w, b)


def _bn_body(y_ref, st_ref, g_ref, bt_ref, xf_ref):
    st = st_ref[...]
    mu = st[0:1, :] * (1.0 / N)
    var = st[1:2, :] * (1.0 / N) - mu * mu
    xf_ref[...] = ((y_ref[...] - mu) * jax.lax.rsqrt(var + 1e-5) * g_ref[...]
                   + bt_ref[...])


def _bn_call(y, st, gamma, beta):
    return pl.pallas_call(
        _bn_body,
        grid=(GRID_N,),
        in_specs=[
            pl.BlockSpec((R, EMB), lambda i: (i, 0)),
            pl.BlockSpec((2, EMB), lambda i: (0, 0)),
            pl.BlockSpec((1, EMB), lambda i: (0, 0)),
            pl.BlockSpec((1, EMB), lambda i: (0, 0)),
        ],
        out_specs=pl.BlockSpec((R, EMB), lambda i: (i, 0)),
        out_shape=jax.ShapeDtypeStruct((N, EMB), jnp.float32),
    )(y, st, gamma, beta)


def _gin0_body(xf_ref, a_ref, w1_ref, b1_ref, w2_ref, b2_ref, eps_ref, hf_ref):
    z = (1.0 + eps_ref[0, 0]) * xf_ref[...] + a_ref[...]
    t = jnp.maximum(_dot(z, w1_ref[...]) + b1_ref[...], 0.0)
    hf_ref[...] = _dot(t, w2_ref[...]) + b2_ref[...]


def _gin0_call(xf, agg, w1, b1, w2, b2, eps):
    return pl.pallas_call(
        _gin0_body,
        grid=(GRID_N,),
        in_specs=[
            pl.BlockSpec((R, EMB), lambda i: (i, 0)),
            pl.BlockSpec((R, EMB), lambda i: (i, 0)),
            pl.BlockSpec((EMB, EMB), lambda i: (0, 0)),
            pl.BlockSpec((1, EMB), lambda i: (0, 0)),
            pl.BlockSpec((EMB, EMB), lambda i: (0, 0)),
            pl.BlockSpec((1, EMB), lambda i: (0, 0)),
            pl.BlockSpec((1, 1), lambda i: (0, 0)),
        ],
        out_specs=pl.BlockSpec((R, EMB), lambda i: (i, 0)),
        out_shape=jax.ShapeDtypeStruct((N, EMB), jnp.float32),
    )(xf, agg, w1, b1, w2, b2, eps)


def _gin_body(xf_ref, hf_ref, ax_ref, ah_ref,
              w1_ref, b1_ref, w2_ref, b2_ref, eps_ref, hfo_ref):
    e = 1.0 + eps_ref[0, 0]
    za = e * xf_ref[...] + ax_ref[...]
    zb = e * hf_ref[...] + ah_ref[...]
    t = jnp.maximum(
        _dot(za, w1_ref[0:EMB, :]) + _dot(zb, w1_ref[EMB:, :]) + b1_ref[...], 0.0)
    hfo_ref[...] = _dot(t, w2_ref[...]) + b2_ref[...]


def _gin_call(xf, hf, ax, ah, w1, b1, w2, b2, eps):
    return pl.pallas_call(
        _gin_body,
        grid=(GRID_N,),
        in_specs=[
            pl.BlockSpec((R, EMB), lambda i: (i, 0)),
            pl.BlockSpec((R, EMB), lambda i: (i, 0)),
            pl.BlockSpec((R, EMB), lambda i: (i, 0)),
            pl.BlockSpec((R, EMB), lambda i: (i, 0)),
            pl.BlockSpec((2 * EMB, EMB), lambda i: (0, 0)),
            pl.BlockSpec((1, EMB), lambda i: (0, 0)),
            pl.BlockSpec((EMB, EMB), lambda i: (0, 0)),
            pl.BlockSpec((1, EMB), lambda i: (0, 0)),
            pl.BlockSpec((1, 1), lambda i: (0, 0)),
        ],
        out_specs=pl.BlockSpec((R, EMB), lambda i: (i, 0)),
        out_shape=jax.ShapeDtypeStruct((N, EMB), jnp.float32),
    )(xf, hf, ax, ah, w1, b1, w2, b2, eps)


def _head_body(xo_ref, w0_ref, b0_ref, w1_ref, b1_ref, w2_ref, b2_ref,
               m_ref, s_ref, l_ref):
    xo = xo_ref[...]
    o1 = _dot(xo, w0_ref[...]) + b0_ref[...]
    o2 = _dot(o1, w1_ref[0:EMB, :]) + _dot(xo, w1_ref[EMB:, :]) + b1_ref[...]
    o3 = _dot(o2, w2_ref[0:EMB, :]) + _dot(xo, w2_ref[EMB:, :]) + b2_ref[...]
    logits = jnp.where(m_ref[...] > 0.5, o3, -1.0e6)
    m = jnp.max(logits, axis=1, keepdims=True)
    ssum = jnp.sum(jnp.exp(logits - m), axis=1, keepdims=True)
    idx = lax.broadcasted_iota(jnp.int32, logits.shape, 1)
    samp = jnp.min(jnp.where(logits == m, idx, N_ACT), axis=1, keepdims=True)
    s_ref[...] = samp
    l_ref[...] = -jnp.log(ssum)


def _head_call(xo, w0, b0, w1, b1, w2, b2, maskf):
    return pl.pallas_call(
        _head_body,
        grid=(1,),
        in_specs=[
            pl.BlockSpec((B, EMB), lambda i: (0, 0)),
            pl.BlockSpec((EMB, EMB), lambda i: (0, 0)),
            pl.BlockSpec((1, EMB), lambda i: (0, 0)),
            pl.BlockSpec((2 * EMB, EMB), lambda i: (0, 0)),
            pl.BlockSpec((1, EMB), lambda i: (0, 0)),
            pl.BlockSpec((2 * EMB, N_ACT), lambda i: (0, 0)),
            pl.BlockSpec((1, N_ACT), lambda i: (0, 0)),
            pl.BlockSpec((B, N_ACT), lambda i: (0, 0)),
        ],
        out_specs=[
            pl.BlockSpec((B, 1), lambda i: (0, 0)),
            pl.BlockSpec((B, 1), lambda i: (0, 0)),
        ],
        out_shape=[
            jax.ShapeDtypeStruct((B, 1), jnp.int32),
            jax.ShapeDtypeStruct((B, 1), jnp.float32),
        ],
    )(xo, w0, b0, w1, b1, w2, b2, maskf)


@functools.cache
def _get_segsum():
    mesh = plsc.VectorSubcoreMesh(core_axis_name="c", subcore_axis_name="s",
                                  num_cores=NC, num_subcores=NS)
    return functools.partial(
        pl.kernel,
        mesh=mesh,
        out_type=jax.ShapeDtypeStruct((2 * N_OUT_PAD, 128), jnp.float32),
        scratch_types=[
            pltpu.VMEM((SLAB, C), jnp.int32),
            pltpu.VMEM((2 * SLAB, C), jnp.int32),
            pltpu.VMEM((RB, 2 * C, 128), jnp.float32),
            pltpu.VMEM_SHARED((2 * ACC_N, 128), jnp.float32),
            [pltpu.SemaphoreType.DMA] * RB,
        ],
    )(_segsum_body)


def _segsum_body(h_hbm, src_hbm, dstl_hbm, zeros_hbm, out_hbm,
                 src_v, dst_v, rows_v, acc_sh, gsems):
    c = lax.axis_index("c")
    s = lax.axis_index("s")
    zrows = 2 * ACC_N // NS

    for q in range(2):
        grp = c * 2 + q

        pltpu.sync_copy(zeros_hbm, acc_sh.at[pl.ds(s * zrows, zrows)])
        plsc.subcore_barrier()

        @pl.loop(0, NSLAB)
        def _slab(t):
            pltpu.sync_copy(src_hbm.at[grp, s, pl.ds(t * SLAB, SLAB)], src_v)
            pltpu.sync_copy(dstl_hbm.at[grp, s, pl.ds(t * 2 * SLAB, 2 * SLAB)],
                            dst_v)
            pltpu.async_copy(h_hbm.at[src_v.at[0]],
                             rows_v.at[0].reshape(C, EMB), gsems[0])
            pltpu.async_copy(h_hbm.at[src_v.at[1]],
                             rows_v.at[1].reshape(C, EMB), gsems[1])

            @pl.loop(0, SLAB, step=RB)
            def _grp_loop(g):
                for b in range(RB):
                    j = g + b
                    pltpu.make_async_copy(h_hbm.at[src_v.at[j]],
                                          rows_v.at[b].reshape(C, EMB),
                                          gsems[b]).wait()
                    for k in range(2):
                        pltpu.sync_copy(rows_v.at[b, pl.ds(k * C, C)],
                                        acc_sh.at[dst_v.at[2 * j + k]],
                                        add=True)

                    @pl.when(j + RB < SLAB)
                    def _():
                        pltpu.async_copy(h_hbm.at[src_v.at[j + RB]],
                                         rows_v.at[b].reshape(C, EMB),
                                         gsems[b])

        plsc.subcore_barrier()

        pltpu.sync_copy(acc_sh.at[pl.ds(s * 2 * WROWS, 2 * WROWS)],
                        out_hbm.at[pl.ds(2 * (grp * QS) + s * 2 * WROWS,
                                         2 * WROWS)])
        plsc.subcore_barrier()


B_PER_W = B // NW


@functools.cache
def _get_center_gather():
    mesh = plsc.VectorSubcoreMesh(core_axis_name="c", subcore_axis_name="s",
                                  num_cores=NC, num_subcores=NS)
    return functools.partial(
        pl.kernel,
        mesh=mesh,
        out_type=jax.ShapeDtypeStruct((B, EMB), jnp.float32),
        scratch_types=[
            pltpu.VMEM((B_PER_W,), jnp.int32),
            pltpu.VMEM((B_PER_W, EMB), jnp.float32),
            pltpu.SemaphoreType.DMA,
        ],
    )(_center_gather_body)


def _center_gather_body(h_hbm, idx_hbm, out_hbm, idx_v, rows_v, sem):
    wid = lax.axis_index("s") * NC + lax.axis_index("c")
    base = wid * B_PER_W
    pltpu.sync_copy(idx_hbm.at[pl.ds(base, B_PER_W)], idx_v)
    pltpu.async_copy(h_hbm.at[idx_v], rows_v, sem).wait()
    pltpu.sync_copy(rows_v, out_hbm.at[pl.ds(base, B_PER_W)])


def _segsum_kernel(h, src_r, dstl_r, zeros_init):
    out = _get_segsum()(h, src_r, dstl_r, zeros_init)
    return out.reshape(N_OUT_PAD, EMB)


def _center_gather_kernel(h, idx):
    return _get_center_gather()(h, idx)


def kernel(x, edge_index, center_node_index, mask, params):
    src = edge_index[0]
    dst = edge_index[1]

    grp = dst // QS
    order = jnp.argsort(grp, stable=True)
    counts = jnp.sum(jax.nn.one_hot(grp, NQ, dtype=jnp.int32), axis=0)
    starts = jnp.concatenate(
        [jnp.zeros((1,), jnp.int32), jnp.cumsum(counts)[:-1]])
    esrc = src[order]
    edstl = (dst - grp * QS)[order]
    slots = jnp.arange(NQ * CAP, dtype=jnp.int32)
    sg = slots // CAP
    so = slots % CAP
    valid = so < counts[sg]
    gi = jnp.clip(starts[sg] + so, 0, E - 1)
    garbage = QS + (slots % (ACC_N - QS))
    src_flat = jnp.where(valid, esrc[gi], 0)
    dstl = jnp.where(valid, edstl[gi], garbage)
    src_r = src_flat.reshape(NQ, NS, NCHUNK_Q, C)
    dstl_r = (jnp.stack([2 * dstl, 2 * dstl + 1], axis=-1)
              .reshape(NQ, NS, 2 * NCHUNK_Q, C))
    zeros_init = jnp.zeros((2 * ACC_N // NS, 128), jnp.float32)

    p = params
    be = p["embed"]["b"].reshape(1, EMB)
    y, st = _embed_call(x, p["embed"]["W"], be)
    xf = _bn_call(y, st, p["bn"]["gamma"].reshape(1, EMB),
                  p["bn"]["beta"].reshape(1, EMB))

    agg_x = _segsum_kernel(xf, src_r, dstl_r, zeros_init)

    g0 = p["gins"][0]
    h0f = _gin0_call(xf, agg_x[:N],
                     g0["lin1"]["W"], g0["lin1"]["b"].reshape(1, EMB),
                     g0["lin2"]["W"], g0["lin2"]["b"].reshape(1, EMB),
                     g0["eps"].reshape(1, 1))

    agg_h0 = _segsum_kernel(h0f, src_r, dstl_r, zeros_init)

    g1 = p["gins"][1]
    h1f = _gin_call(xf, h0f, agg_x[:N], agg_h0[:N],
                    g1["lin1"]["W"], g1["lin1"]["b"].reshape(1, EMB),
                    g1["lin2"]["W"], g1["lin2"]["b"].reshape(1, EMB),
                    g1["eps"].reshape(1, 1))

    agg_h1 = _segsum_kernel(h1f, src_r, dstl_r, zeros_init)

    g2 = p["gins"][2]
    h2f = _gin_call(xf, h1f, agg_x[:N], agg_h1[:N],
                    g2["lin1"]["W"], g2["lin1"]["b"].reshape(1, EMB),
                    g2["lin2"]["W"], g2["lin2"]["b"].reshape(1, EMB),
                    g2["eps"].reshape(1, 1))

    xo = _center_gather_kernel(h2f, center_node_index)

    o0, o1, o2 = p["outs"]
    s2d, l2d = _head_call(xo, o0["W"], o0["b"].reshape(1, EMB),
                          o1["W"], o1["b"].reshape(1, EMB),
                          o2["W"], o2["b"].reshape(1, N_ACT),
                          mask.astype(jnp.float32))
    return s2d[:, 0], l2d[:, 0]
